# baseline (speedup 1.0000x reference)
import functools

import jax
import jax.numpy as jnp
import numpy as np
from jax import lax
from jax.experimental import pallas as pl
from jax.experimental.pallas import tpu as pltpu
from jax.experimental.pallas import tpu_sc as plsc

N_NODES = 10000
N_EDGES = 320000
D = 128
NUM_HEADS = 4
HP = 8

NB = 256
NBLOCK = (N_NODES + NB - 1) // NB
N_PAD = NBLOCK * NB
T = 1024

SC_CH = 128
SC_CPW = 79
SC_NW = 32
E_PAD = SC_NW * SC_CH * SC_CPW


def _sc_gather_body(table_hbm, idx_hbm, out_hbm, idx_v, rows_v,
                    semi, semg, semo):
    nc = 2
    wid = lax.axis_index("s") * nc + lax.axis_index("c")
    base = wid * (SC_CH * SC_CPW)

    def idx_copy(j, slot):
        return pltpu.make_async_copy(
            idx_hbm.at[pl.ds(base + j * SC_CH, SC_CH)],
            idx_v.at[slot], semi.at[slot])

    def gather(slot):
        return pltpu.make_async_copy(
            table_hbm.at[idx_v.at[slot]], rows_v.at[lax.rem(slot, 2)],
            semg.at[lax.rem(slot, 2)])

    def out_copy(j, slot):
        return pltpu.make_async_copy(
            rows_v.at[slot], out_hbm.at[pl.ds(base + j * SC_CH, SC_CH)],
            semo.at[slot])

    idx_copy(0, 0).start()
    idx_copy(1, 1).start()

    def step(j, carry):
        s2 = lax.rem(j, 2)
        s4 = lax.rem(j, 4)
        idx_copy(j, s4).wait()

        @pl.when(j >= 2)
        def _():
            out_copy(j - 2, s2).wait()

        gather(s4).start()

        @pl.when(j + 2 < SC_CPW)
        def _():
            idx_copy(j + 2, lax.rem(j + 2, 4)).start()

        @pl.when(j >= 1)
        def _():
            gather(lax.rem(j - 1, 4)).wait()
            out_copy(j - 1, lax.rem(j - 1, 2)).start()

        return carry

    lax.fori_loop(0, SC_CPW, step, 0)
    gather(lax.rem(SC_CPW - 1, 4)).wait()
    out_copy(SC_CPW - 1, lax.rem(SC_CPW - 1, 2)).start()
    out_copy(SC_CPW - 2, lax.rem(SC_CPW - 2, 2)).wait()
    out_copy(SC_CPW - 1, lax.rem(SC_CPW - 1, 2)).wait()


def _sc_scatter_body(rows_hbm, idx_hbm, out_hbm, idx_v, rows_v,
                     semi, semr, semo):
    nc = 2
    wid = lax.axis_index("s") * nc + lax.axis_index("c")
    base = wid * (SC_CH * SC_CPW)

    def idx_copy(j, slot):
        return pltpu.make_async_copy(
            idx_hbm.at[pl.ds(base + j * SC_CH, SC_CH)],
            idx_v.at[slot], semi.at[slot])

    def row_copy(j, slot):
        return pltpu.make_async_copy(
            rows_hbm.at[pl.ds(base + j * SC_CH, SC_CH)],
            rows_v.at[slot], semr.at[slot])

    def scat(j):
        s2 = lax.rem(j, 2)
        return pltpu.make_async_copy(
            rows_v.at[s2], out_hbm.at[idx_v.at[lax.rem(j, 4)]],
            semo.at[s2])

    idx_copy(0, 0).start()
    idx_copy(1, 1).start()
    row_copy(0, 0).start()

    def step(j, carry):
        s2 = lax.rem(j, 2)
        s4 = lax.rem(j, 4)
        idx_copy(j, s4).wait()
        row_copy(j, s2).wait()
        scat(j).start()

        @pl.when(j >= 1)
        def _():
            scat(j - 1).wait()

        @pl.when(j + 1 < SC_CPW)
        def _():
            row_copy(j + 1, 1 - s2).start()

        @pl.when(j + 2 < SC_CPW)
        def _():
            idx_copy(j + 2, lax.rem(j + 2, 4)).start()

        return carry

    lax.fori_loop(0, SC_CPW, step, 0)
    scat(SC_CPW - 1).wait()


def _sc_scatter(rows, idx):
    mesh = plsc.VectorSubcoreMesh(core_axis_name="c", subcore_axis_name="s")
    fn = pl.kernel(
        _sc_scatter_body,
        out_type=jax.ShapeDtypeStruct((E_PAD, D), jnp.float32),
        mesh=mesh,
        scratch_types=[
            pltpu.VMEM((4, SC_CH), jnp.int32),
            pltpu.VMEM((2, SC_CH, D), jnp.float32),
            pltpu.SemaphoreType.DMA((4,)),
            pltpu.SemaphoreType.DMA((2,)),
            pltpu.SemaphoreType.DMA((2,)),
        ],
    )
    return fn(rows, idx)


def _sc_gather(table, idx):
    mesh = plsc.VectorSubcoreMesh(core_axis_name="c", subcore_axis_name="s")
    fn = pl.kernel(
        _sc_gather_body,
        out_type=jax.ShapeDtypeStruct((E_PAD, D), jnp.float32),
        mesh=mesh,
        scratch_types=[
            pltpu.VMEM((4, SC_CH), jnp.int32),
            pltpu.VMEM((2, SC_CH, D), jnp.float32),
            pltpu.SemaphoreType.DMA((4,)),
            pltpu.SemaphoreType.DMA((2,)),
            pltpu.SemaphoreType.DMA((2,)),
        ],
    )
    return fn(table, idx)


_SQRT2 = float(np.sqrt(2.0))


def _gelu(x):
    return 0.5 * x * (1.0 + lax.erf(x / _SQRT2))


def _dgt0(a, b):
    return lax.dot_general(a, b, (((0,), (0,)), ((), ())),
                           preferred_element_type=jnp.float32)


def _mm(a, b):
    return jnp.dot(a, b, preferred_element_type=jnp.float32)


def _bmm(a, b):
    return jnp.dot(a.astype(jnp.bfloat16), b,
                   preferred_element_type=jnp.float32)


def _layer_body(rs_ref, s_ref, z_ref, ssrc_ref, dl_ref,
                ws_pack, wd_pack, wzc, be1, we2, be2,
                wzn_pack, ba1, wa2, ba2, wa3, ba3,
                bv1, wv2, bv2, wv3, bv3,
                wao, bao, wf1, bf1, wf2, bf2,
                snew_ref, znew_ref,
                z_v, ssrc_v, dl_v, znew_v, U, den, mx,
                semz, sems, semd, semo):
    b = pl.program_id(0)
    rs0 = rs_ref[b]
    n_e = rs_ref[b + 1] - rs0
    nch = (n_e + T - 1) // T

    U[...] = jnp.zeros_like(U)
    den[...] = jnp.zeros_like(den)
    mx[...] = jnp.full_like(mx, -1e30)
    s_blk = s_ref[...]
    s_blk16 = s_blk.astype(jnp.bfloat16)

    def in_copies(k, slot):
        start = rs0 + k * T
        return (
            pltpu.make_async_copy(z_ref.at[pl.ds(start, T)],
                                  z_v.at[slot], semz.at[slot]),
            pltpu.make_async_copy(ssrc_ref.at[pl.ds(start, T)],
                                  ssrc_v.at[slot], sems.at[slot]),
            pltpu.make_async_copy(dl_ref.at[pl.ds(start, T)],
                                  dl_v.at[slot], semd.at[slot]),
        )

    def out_copy(k, slot):
        return pltpu.make_async_copy(
            znew_v.at[slot], znew_ref.at[pl.ds(rs0 + k * T, T)],
            semo.at[slot])

    @pl.when(nch > 0)
    def _():
        for c in in_copies(0, 0):
            c.start()

    def chunk(k, carry):
        slot = lax.rem(k, 2)
        nslot = 1 - slot
        @pl.when(k + 1 < nch)
        def _():
            for c in in_copies(k + 1, nslot):
                c.start()
        for c in in_copies(k, slot):
            c.wait()

        zc = z_v[slot]
        ss = ssrc_v[slot]
        dl = dl_v[slot]
        eidx = lax.broadcasted_iota(jnp.int32, (T, 1), 0) + k * T
        valid = eidx < n_e
        hit = jnp.logical_and(
            dl == lax.broadcasted_iota(jnp.int32, (T, NB), 1), valid)
        Pt = hit.astype(jnp.bfloat16)

        sd = _bmm(Pt, s_blk16)

        ss_out = _bmm(ss, ws_pack[...])
        sd_out = _bmm(sd, wd_pack[...])
        zc_out = _bmm(zc, wzc[...])

        e1 = _gelu(ss_out[:, 0:D] + sd_out[:, 0:D] + zc_out + be1[...])
        zn = zc + _bmm(e1, we2[...]) + be2[...]
        @pl.when(k >= 2)
        def _():
            out_copy(k - 2, slot).wait()
        znew_v[slot] = zn
        out_copy(k, slot).start()

        zn_out = _bmm(zn, wzn_pack[...])

        a1 = _gelu(sd_out[:, D:2 * D] + ss_out[:, D:2 * D]
                   + zn_out[:, 0:D] + ba1[...])
        a2 = _gelu(_bmm(a1, wa2[...]) + ba2[...])
        aw = _bmm(a2, wa3[...]) + ba3[...]
        v1 = _gelu(ss_out[:, 2 * D:3 * D] + zn_out[:, D:2 * D] + bv1[...])
        v2 = _gelu(_bmm(v1, wv2[...]) + bv2[...])
        av = _bmm(v2, wv3[...]) + bv3[...]

        awm = jnp.where(valid, aw, -1e30)
        cmax = jnp.max(awm, axis=0, keepdims=True)
        mold = mx[...]
        mnew = jnp.maximum(mold, cmax)
        scale = jnp.exp(mold - mnew)
        mx[...] = mnew
        ew = jnp.exp(awm - mnew)
        den[...] = den[...] * scale + _dgt0(Pt.astype(jnp.float32), ew)
        ew16 = ew.astype(jnp.bfloat16)
        ptw = jnp.concatenate(
            [Pt * ew16[:, h:h + 1] for h in range(NUM_HEADS)], axis=1)
        srow = jnp.concatenate(
            [jnp.broadcast_to(scale[0:1, h:h + 1], (NB, 1))
             for h in range(NUM_HEADS)], axis=0)
        U[...] = U[...] * srow + _dgt0(ptw, av.astype(jnp.bfloat16))
        return carry

    lax.fori_loop(0, nch, chunk, 0)

    @pl.when(nch >= 2)
    def _():
        out_copy(nch - 2, lax.rem(nch, 2)).wait()

    @pl.when(nch >= 1)
    def _():
        out_copy(nch - 1, lax.rem(nch - 1, 2)).wait()

    dd = den[...]
    attn = [U[h * NB:(h + 1) * NB, :] / (dd[:, h:h + 1] + 1e-16)
            for h in range(NUM_HEADS)]
    attn_cat = jnp.concatenate(attn, axis=1)
    so = s_blk + _bmm(attn_cat, wao[...]) + bao[...]
    f1 = _gelu(_bmm(so, wf1[...]) + bf1[...])
    snew_ref[...] = so + _bmm(f1, wf2[...]) + bf2[...]


_N_W = 22


def _layer_specs():
    in_specs = [
        pl.BlockSpec(memory_space=pltpu.SMEM),
        pl.BlockSpec((NB, D), lambda b: (b, 0)),
        pl.BlockSpec(memory_space=pl.ANY),
        pl.BlockSpec(memory_space=pl.ANY),
        pl.BlockSpec(memory_space=pl.ANY),
    ]
    out_specs = [
        pl.BlockSpec((NB, D), lambda b: (b, 0)),
        pl.BlockSpec(memory_space=pl.ANY),
    ]
    out_shapes = [
        jax.ShapeDtypeStruct((N_PAD, D), jnp.float32),
        jax.ShapeDtypeStruct((E_PAD, D), jnp.float32),
    ]
    scratch_shapes = [
        pltpu.VMEM((2, T, D), jnp.float32),
        pltpu.VMEM((2, T, D), jnp.float32),
        pltpu.VMEM((2, T, 1), jnp.int32),
        pltpu.VMEM((2, T, D), jnp.float32),
        pltpu.VMEM((NUM_HEADS * NB, D), jnp.float32),
        pltpu.VMEM((NB, HP), jnp.float32),
        pltpu.VMEM((1, HP), jnp.float32),
        pltpu.SemaphoreType.DMA((2,)),
        pltpu.SemaphoreType.DMA((2,)),
        pltpu.SemaphoreType.DMA((2,)),
        pltpu.SemaphoreType.DMA((2,)),
    ]
    return in_specs, out_specs, out_shapes, scratch_shapes


def _layer_call(row_start, s_pad, z_s, ssrc, dstloc, weights, interpret=False):
    in_specs, out_specs, out_shapes, scratch_shapes = _layer_specs()
    full_vmem = [pl.BlockSpec(w.shape, lambda b, _n=w.ndim: (0,) * _n)
                 for w in weights]
    fn = pl.pallas_call(
        _layer_body,
        grid=(NBLOCK,),
        in_specs=in_specs + full_vmem,
        out_specs=out_specs,
        out_shape=out_shapes,
        scratch_shapes=scratch_shapes,
        compiler_params=pltpu.CompilerParams(
            dimension_semantics=("arbitrary",)),
        interpret=interpret,
    )
    return fn(row_start, s_pad, z_s, ssrc, dstloc, *weights)


def _fold(p):
    g = 1.0 / np.sqrt(1.0 + 1e-5)

    def bnfold(w, bvec, gamma, beta):
        sc = gamma * g
        return w * sc[None, :], (bvec * sc + beta)[None, :]

    (ew1, eb1), (ew2, eb2) = p['eff']
    ew2f, eb2f = bnfold(ew2, eb2, p['eff_bn'][0], p['eff_bn'][1])
    (aw1, ab1), (aw2, ab2), (aw3, ab3) = p['aw']
    aw3p = jnp.zeros((D, HP), jnp.float32).at[:, :NUM_HEADS].set(aw3)
    ab3p = jnp.zeros((HP,), jnp.float32).at[:NUM_HEADS].set(ab3)
    (vw1, vb1), (vw2, vb2), (vw3, vb3) = p['av']
    ow, ob = p['ao']
    owf, obf = bnfold(ow, ob, p['ao_bn'][0], p['ao_bn'][1])
    (fw1, fb1), (fw2, fb2) = p['ffn']
    fw2f, fb2f = bnfold(fw2, fb2, p['ffn_bn'][0], p['ffn_bn'][1])
    ws_pack = jnp.concatenate([ew1[0:D], aw1[D:2 * D], vw1[0:D]], axis=1)
    wd_pack = jnp.concatenate([ew1[D:2 * D], aw1[0:D]], axis=1)
    wzc = ew1[2 * D:3 * D]
    wzn_pack = jnp.concatenate([aw1[2 * D:3 * D], vw1[D:2 * D]], axis=1)
    b16 = lambda w: w.astype(jnp.bfloat16)
    return [
        b16(ws_pack), b16(wd_pack), b16(wzc), eb1[None, :],
        b16(ew2f), eb2f,
        b16(wzn_pack), ab1[None, :], b16(aw2), ab2[None, :],
        b16(aw3p), ab3p[None, :],
        vb1[None, :], b16(vw2), vb2[None, :], b16(vw3), vb3[None, :],
        b16(owf), obf, b16(fw1), fb1[None, :], b16(fw2f), fb2f,
    ]


def kernel(s, z, edge_idx, params):
    src = edge_idx[0]
    dst = edge_idx[1]

    eiota = jnp.arange(N_EDGES, dtype=jnp.int32)
    dst_s, src_s, perm = lax.sort((dst, src, eiota), num_keys=1)
    pad_i = jnp.zeros((E_PAD - N_EDGES,), jnp.int32)
    src_p = jnp.concatenate([src_s, pad_i])
    perm_p = jnp.concatenate([perm, pad_i])
    perm_scat = jnp.concatenate(
        [perm, jnp.arange(N_EDGES, E_PAD, dtype=jnp.int32)])
    dstloc = jnp.concatenate([(dst_s % NB).astype(jnp.int32), pad_i])
    dstloc = dstloc.reshape(E_PAD, 1)
    bounds = jnp.arange(0, N_PAD + NB, NB, dtype=jnp.int32)
    row_start = jnp.searchsorted(dst_s, bounds).astype(jnp.int32)

    s_pad = jnp.zeros((N_PAD, D), jnp.float32).at[:N_NODES, :].set(s)
    z_s = _sc_gather(z, perm_p)

    for p in params:
        weights = _fold(p)
        ssrc = _sc_gather(s_pad, src_p)
        s_pad, z_s = _layer_call(row_start, s_pad, z_s, ssrc, dstloc, weights)

    z_out = _sc_scatter(z_s, perm_scat)[:N_EDGES]
    return (s_pad[:N_NODES], z_out)


# speedup vs baseline: 1.0748x; 1.0748x over previous
import functools

import jax
import jax.numpy as jnp
import numpy as np
from jax import lax
from jax.experimental import pallas as pl
from jax.experimental.pallas import tpu as pltpu
from jax.experimental.pallas import tpu_sc as plsc

N_NODES = 10000
N_EDGES = 320000
D = 128
NUM_HEADS = 4
HP = 8

NB = 128
NBLOCK = (N_NODES + NB - 1) // NB
N_PAD = NBLOCK * NB
T = 1024

SC_CH = 128
SC_CPW = 79
SC_NW = 32
E_PAD = SC_NW * SC_CH * SC_CPW


def _sc_gather_body(table_hbm, idx_hbm, out_hbm, idx_v, rows_v,
                    semi, semg, semo):
    nc = 2
    wid = lax.axis_index("s") * nc + lax.axis_index("c")
    base = wid * (SC_CH * SC_CPW)

    def idx_copy(j, slot):
        return pltpu.make_async_copy(
            idx_hbm.at[pl.ds(base + j * SC_CH, SC_CH)],
            idx_v.at[slot], semi.at[slot])

    def gather(slot):
        return pltpu.make_async_copy(
            table_hbm.at[idx_v.at[slot]], rows_v.at[lax.rem(slot, 2)],
            semg.at[lax.rem(slot, 2)])

    def out_copy(j, slot):
        return pltpu.make_async_copy(
            rows_v.at[slot], out_hbm.at[pl.ds(base + j * SC_CH, SC_CH)],
            semo.at[slot])

    idx_copy(0, 0).start()
    idx_copy(1, 1).start()

    def step(j, carry):
        s2 = lax.rem(j, 2)
        s4 = lax.rem(j, 4)
        idx_copy(j, s4).wait()

        @pl.when(j >= 2)
        def _():
            out_copy(j - 2, s2).wait()

        gather(s4).start()

        @pl.when(j + 2 < SC_CPW)
        def _():
            idx_copy(j + 2, lax.rem(j + 2, 4)).start()

        @pl.when(j >= 1)
        def _():
            gather(lax.rem(j - 1, 4)).wait()
            out_copy(j - 1, lax.rem(j - 1, 2)).start()

        return carry

    lax.fori_loop(0, SC_CPW, step, 0)
    gather(lax.rem(SC_CPW - 1, 4)).wait()
    out_copy(SC_CPW - 1, lax.rem(SC_CPW - 1, 2)).start()
    out_copy(SC_CPW - 2, lax.rem(SC_CPW - 2, 2)).wait()
    out_copy(SC_CPW - 1, lax.rem(SC_CPW - 1, 2)).wait()


def _sc_scatter_body(rows_hbm, idx_hbm, out_hbm, idx_v, rows_v,
                     semi, semr, semo):
    nc = 2
    wid = lax.axis_index("s") * nc + lax.axis_index("c")
    base = wid * (SC_CH * SC_CPW)

    def idx_copy(j, slot):
        return pltpu.make_async_copy(
            idx_hbm.at[pl.ds(base + j * SC_CH, SC_CH)],
            idx_v.at[slot], semi.at[slot])

    def row_copy(j, slot):
        return pltpu.make_async_copy(
            rows_hbm.at[pl.ds(base + j * SC_CH, SC_CH)],
            rows_v.at[slot], semr.at[slot])

    def scat(j):
        s2 = lax.rem(j, 2)
        return pltpu.make_async_copy(
            rows_v.at[s2], out_hbm.at[idx_v.at[lax.rem(j, 4)]],
            semo.at[s2])

    idx_copy(0, 0).start()
    idx_copy(1, 1).start()
    row_copy(0, 0).start()

    def step(j, carry):
        s2 = lax.rem(j, 2)
        s4 = lax.rem(j, 4)
        idx_copy(j, s4).wait()
        row_copy(j, s2).wait()
        scat(j).start()

        @pl.when(j >= 1)
        def _():
            scat(j - 1).wait()

        @pl.when(j + 1 < SC_CPW)
        def _():
            row_copy(j + 1, 1 - s2).start()

        @pl.when(j + 2 < SC_CPW)
        def _():
            idx_copy(j + 2, lax.rem(j + 2, 4)).start()

        return carry

    lax.fori_loop(0, SC_CPW, step, 0)
    scat(SC_CPW - 1).wait()


def _sc_scatter(rows, idx):
    mesh = plsc.VectorSubcoreMesh(core_axis_name="c", subcore_axis_name="s")
    fn = pl.kernel(
        _sc_scatter_body,
        out_type=jax.ShapeDtypeStruct((E_PAD, D), jnp.float32),
        mesh=mesh,
        scratch_types=[
            pltpu.VMEM((4, SC_CH), jnp.int32),
            pltpu.VMEM((2, SC_CH, D), jnp.float32),
            pltpu.SemaphoreType.DMA((4,)),
            pltpu.SemaphoreType.DMA((2,)),
            pltpu.SemaphoreType.DMA((2,)),
        ],
    )
    return fn(rows, idx)


def _sc_gather(table, idx):
    mesh = plsc.VectorSubcoreMesh(core_axis_name="c", subcore_axis_name="s")
    fn = pl.kernel(
        _sc_gather_body,
        out_type=jax.ShapeDtypeStruct((E_PAD, D), jnp.float32),
        mesh=mesh,
        scratch_types=[
            pltpu.VMEM((4, SC_CH), jnp.int32),
            pltpu.VMEM((2, SC_CH, D), jnp.float32),
            pltpu.SemaphoreType.DMA((4,)),
            pltpu.SemaphoreType.DMA((2,)),
            pltpu.SemaphoreType.DMA((2,)),
        ],
    )
    return fn(table, idx)


_SQRT2 = float(np.sqrt(2.0))


def _gelu(x):
    return 0.5 * x * (1.0 + lax.erf(x / _SQRT2))


def _dgt0(a, b):
    return lax.dot_general(a, b, (((0,), (0,)), ((), ())),
                           preferred_element_type=jnp.float32)


def _mm(a, b):
    return jnp.dot(a, b, preferred_element_type=jnp.float32)


def _bmm(a, b):
    return jnp.dot(a.astype(jnp.bfloat16), b,
                   preferred_element_type=jnp.float32)


def _layer_body(rs_ref, s_ref, z_ref, ssrc_ref, dl_ref,
                ws_pack, wd_pack, wzc, be1, we2, be2,
                wzn_pack, ba1, wa2, ba2, wa3, ba3,
                bv1, wv2, bv2, wv3, bv3,
                wao, bao, wf1, bf1, wf2, bf2,
                snew_ref, znew_ref,
                z_v, ssrc_v, dl_v, znew_v, U, den, mx,
                semz, sems, semd, semo):
    b = pl.program_id(0)
    rs0 = rs_ref[b]
    n_e = rs_ref[b + 1] - rs0
    nch = (n_e + T - 1) // T

    U[...] = jnp.zeros_like(U)
    den[...] = jnp.zeros_like(den)
    mx[...] = jnp.full_like(mx, -1e30)
    s_blk = s_ref[...]
    s_blk16 = s_blk.astype(jnp.bfloat16)

    def in_copies(k, slot):
        start = rs0 + k * T
        return (
            pltpu.make_async_copy(z_ref.at[pl.ds(start, T)],
                                  z_v.at[slot], semz.at[slot]),
            pltpu.make_async_copy(ssrc_ref.at[pl.ds(start, T)],
                                  ssrc_v.at[slot], sems.at[slot]),
            pltpu.make_async_copy(dl_ref.at[pl.ds(start, T)],
                                  dl_v.at[slot], semd.at[slot]),
        )

    def out_copy(k, slot):
        return pltpu.make_async_copy(
            znew_v.at[slot], znew_ref.at[pl.ds(rs0 + k * T, T)],
            semo.at[slot])

    @pl.when(jnp.logical_and(b == 0, nch > 0))
    def _():
        for c in in_copies(0, 0):
            c.start()

    def chunk(k, carry):
        slot = lax.rem(k, 2)
        nslot = 1 - slot
        @pl.when(k + 1 < nch)
        def _():
            for c in in_copies(k + 1, nslot):
                c.start()
        for c in in_copies(k, slot):
            c.wait()

        zc = z_v[slot]
        ss = ssrc_v[slot]
        dl = dl_v[slot]
        eidx = lax.broadcasted_iota(jnp.int32, (T, 1), 0) + k * T
        valid = eidx < n_e
        hit = jnp.logical_and(
            dl == lax.broadcasted_iota(jnp.int32, (T, NB), 1), valid)
        Pt = hit.astype(jnp.bfloat16)

        sd = _bmm(Pt, s_blk16)

        ss_out = _bmm(ss, ws_pack[...])
        sd_out = _bmm(sd, wd_pack[...])
        zc_out = _bmm(zc, wzc[...])

        e1 = _gelu(ss_out[:, 0:D] + sd_out[:, 0:D] + zc_out + be1[...])
        zn = zc + _bmm(e1, we2[...]) + be2[...]
        @pl.when(k >= 2)
        def _():
            out_copy(k - 2, slot).wait()
        znew_v[slot] = zn
        out_copy(k, slot).start()

        zn_out = _bmm(zn, wzn_pack[...])

        a1 = _gelu(sd_out[:, D:2 * D] + ss_out[:, D:2 * D]
                   + zn_out[:, 0:D] + ba1[...])
        a2 = _gelu(_bmm(a1, wa2[...]) + ba2[...])
        aw = _bmm(a2, wa3[...]) + ba3[...]
        v1 = _gelu(ss_out[:, 2 * D:3 * D] + zn_out[:, D:2 * D] + bv1[...])
        v2 = _gelu(_bmm(v1, wv2[...]) + bv2[...])
        av = _bmm(v2, wv3[...]) + bv3[...]

        awm = jnp.where(valid, aw, -1e30)
        cmax = jnp.max(awm, axis=0, keepdims=True)
        mold = mx[...]
        mnew = jnp.maximum(mold, cmax)
        scale = jnp.exp(mold - mnew)
        mx[...] = mnew
        ew = jnp.exp(awm - mnew)
        ew16 = ew.astype(jnp.bfloat16)
        den[...] = den[...] * scale + _dgt0(Pt, ew16)
        ptw = jnp.concatenate(
            [Pt * ew16[:, h:h + 1] for h in range(NUM_HEADS)], axis=1)
        srow = jnp.concatenate(
            [jnp.broadcast_to(scale[0:1, h:h + 1], (NB, 1))
             for h in range(NUM_HEADS)], axis=0)
        U[...] = U[...] * srow + _dgt0(ptw, av.astype(jnp.bfloat16))
        return carry

    lax.fori_loop(0, nch, chunk, 0)

    @pl.when(b + 1 < NBLOCK)
    def _():
        rs0n = rs_ref[b + 1]

        @pl.when(rs_ref[b + 2] > rs0n)
        def _():
            start = rs0n
            pltpu.make_async_copy(z_ref.at[pl.ds(start, T)],
                                  z_v.at[0], semz.at[0]).start()
            pltpu.make_async_copy(ssrc_ref.at[pl.ds(start, T)],
                                  ssrc_v.at[0], sems.at[0]).start()
            pltpu.make_async_copy(dl_ref.at[pl.ds(start, T)],
                                  dl_v.at[0], semd.at[0]).start()

    @pl.when(nch >= 2)
    def _():
        out_copy(nch - 2, lax.rem(nch, 2)).wait()

    @pl.when(nch >= 1)
    def _():
        out_copy(nch - 1, lax.rem(nch - 1, 2)).wait()

    dd = den[...]
    attn = [U[h * NB:(h + 1) * NB, :] / (dd[:, h:h + 1] + 1e-16)
            for h in range(NUM_HEADS)]
    attn_cat = jnp.concatenate(attn, axis=1)
    so = s_blk + _bmm(attn_cat, wao[...]) + bao[...]
    f1 = _gelu(_bmm(so, wf1[...]) + bf1[...])
    snew_ref[...] = so + _bmm(f1, wf2[...]) + bf2[...]


_N_W = 22


def _layer_specs():
    in_specs = [
        pl.BlockSpec(memory_space=pltpu.SMEM),
        pl.BlockSpec((NB, D), lambda b: (b, 0)),
        pl.BlockSpec(memory_space=pl.ANY),
        pl.BlockSpec(memory_space=pl.ANY),
        pl.BlockSpec(memory_space=pl.ANY),
    ]
    out_specs = [
        pl.BlockSpec((NB, D), lambda b: (b, 0)),
        pl.BlockSpec(memory_space=pl.ANY),
    ]
    out_shapes = [
        jax.ShapeDtypeStruct((N_PAD, D), jnp.float32),
        jax.ShapeDtypeStruct((E_PAD, D), jnp.float32),
    ]
    scratch_shapes = [
        pltpu.VMEM((2, T, D), jnp.float32),
        pltpu.VMEM((2, T, D), jnp.float32),
        pltpu.VMEM((2, T, 1), jnp.int32),
        pltpu.VMEM((2, T, D), jnp.float32),
        pltpu.VMEM((NUM_HEADS * NB, D), jnp.float32),
        pltpu.VMEM((NB, HP), jnp.float32),
        pltpu.VMEM((1, HP), jnp.float32),
        pltpu.SemaphoreType.DMA((2,)),
        pltpu.SemaphoreType.DMA((2,)),
        pltpu.SemaphoreType.DMA((2,)),
        pltpu.SemaphoreType.DMA((2,)),
    ]
    return in_specs, out_specs, out_shapes, scratch_shapes


def _layer_call(row_start, s_pad, z_s, ssrc, dstloc, weights, interpret=False):
    in_specs, out_specs, out_shapes, scratch_shapes = _layer_specs()
    full_vmem = [pl.BlockSpec(w.shape, lambda b, _n=w.ndim: (0,) * _n)
                 for w in weights]
    fn = pl.pallas_call(
        _layer_body,
        grid=(NBLOCK,),
        in_specs=in_specs + full_vmem,
        out_specs=out_specs,
        out_shape=out_shapes,
        scratch_shapes=scratch_shapes,
        compiler_params=pltpu.CompilerParams(
            dimension_semantics=("arbitrary",)),
        interpret=interpret,
    )
    return fn(row_start, s_pad, z_s, ssrc, dstloc, *weights)


def _fold(p):
    g = 1.0 / np.sqrt(1.0 + 1e-5)

    def bnfold(w, bvec, gamma, beta):
        sc = gamma * g
        return w * sc[None, :], (bvec * sc + beta)[None, :]

    (ew1, eb1), (ew2, eb2) = p['eff']
    ew2f, eb2f = bnfold(ew2, eb2, p['eff_bn'][0], p['eff_bn'][1])
    (aw1, ab1), (aw2, ab2), (aw3, ab3) = p['aw']
    aw3p = jnp.zeros((D, HP), jnp.float32).at[:, :NUM_HEADS].set(aw3)
    ab3p = jnp.zeros((HP,), jnp.float32).at[:NUM_HEADS].set(ab3)
    (vw1, vb1), (vw2, vb2), (vw3, vb3) = p['av']
    ow, ob = p['ao']
    owf, obf = bnfold(ow, ob, p['ao_bn'][0], p['ao_bn'][1])
    (fw1, fb1), (fw2, fb2) = p['ffn']
    fw2f, fb2f = bnfold(fw2, fb2, p['ffn_bn'][0], p['ffn_bn'][1])
    ws_pack = jnp.concatenate([ew1[0:D], aw1[D:2 * D], vw1[0:D]], axis=1)
    wd_pack = jnp.concatenate([ew1[D:2 * D], aw1[0:D]], axis=1)
    wzc = ew1[2 * D:3 * D]
    wzn_pack = jnp.concatenate([aw1[2 * D:3 * D], vw1[D:2 * D]], axis=1)
    b16 = lambda w: w.astype(jnp.bfloat16)
    return [
        b16(ws_pack), b16(wd_pack), b16(wzc), eb1[None, :],
        b16(ew2f), eb2f,
        b16(wzn_pack), ab1[None, :], b16(aw2), ab2[None, :],
        b16(aw3p), ab3p[None, :],
        vb1[None, :], b16(vw2), vb2[None, :], b16(vw3), vb3[None, :],
        b16(owf), obf, b16(fw1), fb1[None, :], b16(fw2f), fb2f,
    ]


def kernel(s, z, edge_idx, params):
    src = edge_idx[0]
    dst = edge_idx[1]

    eiota = jnp.arange(N_EDGES, dtype=jnp.int32)
    dst_s, src_s, perm = lax.sort((dst, src, eiota), num_keys=1)
    pad_i = jnp.zeros((E_PAD - N_EDGES,), jnp.int32)
    src_p = jnp.concatenate([src_s, pad_i])
    perm_p = jnp.concatenate([perm, pad_i])
    perm_scat = jnp.concatenate(
        [perm, jnp.arange(N_EDGES, E_PAD, dtype=jnp.int32)])
    dstloc = jnp.concatenate([(dst_s % NB).astype(jnp.int32), pad_i])
    dstloc = dstloc.reshape(E_PAD, 1)
    bounds = jnp.arange(0, N_PAD + NB, NB, dtype=jnp.int32)
    row_start = jnp.searchsorted(dst_s, bounds).astype(jnp.int32)

    s_pad = jnp.zeros((N_PAD, D), jnp.float32).at[:N_NODES, :].set(s)
    z_s = _sc_gather(z, perm_p)

    for p in params:
        weights = _fold(p)
        ssrc = _sc_gather(s_pad, src_p)
        s_pad, z_s = _layer_call(row_start, s_pad, z_s, ssrc, dstloc, weights)

    z_out = _sc_scatter(z_s, perm_scat)[:N_EDGES]
    return (s_pad[:N_NODES], z_out)


# speedup vs baseline: 1.0774x; 1.0024x over previous
import functools

import jax
import jax.numpy as jnp
import numpy as np
from jax import lax
from jax.experimental import pallas as pl
from jax.experimental.pallas import tpu as pltpu
from jax.experimental.pallas import tpu_sc as plsc

N_NODES = 10000
N_EDGES = 320000
D = 128
NUM_HEADS = 4
HP = 8

NB = 128
NBLOCK = (N_NODES + NB - 1) // NB
N_PAD = NBLOCK * NB
T = 1024

SC_CH = 128
SC_CPW = 79
SC_NW = 32
E_PAD = SC_NW * SC_CH * SC_CPW


def _sc_gather_body(table_hbm, idx_hbm, out_hbm, idx_v, rows_v,
                    semi, semg, semo):
    nc = 2
    wid = lax.axis_index("s") * nc + lax.axis_index("c")
    base = wid * (SC_CH * SC_CPW)

    def idx_copy(j, slot):
        return pltpu.make_async_copy(
            idx_hbm.at[pl.ds(base + j * SC_CH, SC_CH)],
            idx_v.at[slot], semi.at[slot])

    def gather(slot):
        return pltpu.make_async_copy(
            table_hbm.at[idx_v.at[slot]], rows_v.at[lax.rem(slot, 2)],
            semg.at[lax.rem(slot, 2)])

    def out_copy(j, slot):
        return pltpu.make_async_copy(
            rows_v.at[slot], out_hbm.at[pl.ds(base + j * SC_CH, SC_CH)],
            semo.at[slot])

    idx_copy(0, 0).start()
    idx_copy(1, 1).start()

    def step(j, carry):
        s2 = lax.rem(j, 2)
        s4 = lax.rem(j, 4)
        idx_copy(j, s4).wait()

        @pl.when(j >= 2)
        def _():
            out_copy(j - 2, s2).wait()

        gather(s4).start()

        @pl.when(j + 2 < SC_CPW)
        def _():
            idx_copy(j + 2, lax.rem(j + 2, 4)).start()

        @pl.when(j >= 1)
        def _():
            gather(lax.rem(j - 1, 4)).wait()
            out_copy(j - 1, lax.rem(j - 1, 2)).start()

        return carry

    lax.fori_loop(0, SC_CPW, step, 0)
    gather(lax.rem(SC_CPW - 1, 4)).wait()
    out_copy(SC_CPW - 1, lax.rem(SC_CPW - 1, 2)).start()
    out_copy(SC_CPW - 2, lax.rem(SC_CPW - 2, 2)).wait()
    out_copy(SC_CPW - 1, lax.rem(SC_CPW - 1, 2)).wait()


def _sc_scatter_body(rows_hbm, idx_hbm, out_hbm, idx_v, rows_v,
                     semi, semr, semo):
    nc = 2
    wid = lax.axis_index("s") * nc + lax.axis_index("c")
    base = wid * (SC_CH * SC_CPW)

    def idx_copy(j, slot):
        return pltpu.make_async_copy(
            idx_hbm.at[pl.ds(base + j * SC_CH, SC_CH)],
            idx_v.at[slot], semi.at[slot])

    def row_copy(j, slot):
        return pltpu.make_async_copy(
            rows_hbm.at[pl.ds(base + j * SC_CH, SC_CH)],
            rows_v.at[slot], semr.at[slot])

    def scat(j):
        s2 = lax.rem(j, 2)
        return pltpu.make_async_copy(
            rows_v.at[s2], out_hbm.at[idx_v.at[lax.rem(j, 4)]],
            semo.at[s2])

    idx_copy(0, 0).start()
    idx_copy(1, 1).start()
    row_copy(0, 0).start()

    def step(j, carry):
        s2 = lax.rem(j, 2)
        s4 = lax.rem(j, 4)
        idx_copy(j, s4).wait()
        row_copy(j, s2).wait()
        scat(j).start()

        @pl.when(j >= 1)
        def _():
            scat(j - 1).wait()

        @pl.when(j + 1 < SC_CPW)
        def _():
            row_copy(j + 1, 1 - s2).start()

        @pl.when(j + 2 < SC_CPW)
        def _():
            idx_copy(j + 2, lax.rem(j + 2, 4)).start()

        return carry

    lax.fori_loop(0, SC_CPW, step, 0)
    scat(SC_CPW - 1).wait()


def _sc_scatter(rows, idx):
    mesh = plsc.VectorSubcoreMesh(core_axis_name="c", subcore_axis_name="s")
    fn = pl.kernel(
        _sc_scatter_body,
        out_type=jax.ShapeDtypeStruct((E_PAD, D), jnp.float32),
        mesh=mesh,
        scratch_types=[
            pltpu.VMEM((4, SC_CH), jnp.int32),
            pltpu.VMEM((2, SC_CH, D), jnp.float32),
            pltpu.SemaphoreType.DMA((4,)),
            pltpu.SemaphoreType.DMA((2,)),
            pltpu.SemaphoreType.DMA((2,)),
        ],
    )
    return fn(rows, idx)


def _sc_gather(table, idx):
    w = table.shape[1]
    mesh = plsc.VectorSubcoreMesh(core_axis_name="c", subcore_axis_name="s")
    fn = pl.kernel(
        _sc_gather_body,
        out_type=jax.ShapeDtypeStruct((E_PAD, w), table.dtype),
        mesh=mesh,
        scratch_types=[
            pltpu.VMEM((4, SC_CH), jnp.int32),
            pltpu.VMEM((2, SC_CH, w), table.dtype),
            pltpu.SemaphoreType.DMA((4,)),
            pltpu.SemaphoreType.DMA((2,)),
            pltpu.SemaphoreType.DMA((2,)),
        ],
    )
    return fn(table, idx)


_SQRT2 = float(np.sqrt(2.0))


def _gelu(x):
    return 0.5 * x * (1.0 + lax.erf(x / _SQRT2))


def _dgt0(a, b):
    return lax.dot_general(a, b, (((0,), (0,)), ((), ())),
                           preferred_element_type=jnp.float32)


def _mm(a, b):
    return jnp.dot(a, b, preferred_element_type=jnp.float32)


def _bmm(a, b):
    return jnp.dot(a.astype(jnp.bfloat16), b,
                   preferred_element_type=jnp.float32)


def _layer_body(rs_ref, s_ref, z_ref, ssrc_ref, dl_ref,
                ws_pack, wd_pack, wzc, be1, we2, be2,
                wzn_pack, ba1, wa2, ba2, wa3, ba3,
                bv1, wv2, bv2, wv3, bv3,
                wao, bao, wf1, bf1, wf2, bf2,
                snew_ref, znew_ref,
                z_v, ssrc_v, dl_v, znew_v, U, den, mx,
                semz, sems, semd, semo):
    b = pl.program_id(0)
    rs0 = rs_ref[b]
    n_e = rs_ref[b + 1] - rs0
    nch = (n_e + T - 1) // T

    U[...] = jnp.zeros_like(U)
    den[...] = jnp.zeros_like(den)
    mx[...] = jnp.full_like(mx, -1e30)
    s_blk = s_ref[...]
    s_blk16 = s_blk.astype(jnp.bfloat16)

    def in_copies(k, slot):
        start = rs0 + k * T
        return (
            pltpu.make_async_copy(z_ref.at[pl.ds(start, T)],
                                  z_v.at[slot], semz.at[slot]),
            pltpu.make_async_copy(ssrc_ref.at[pl.ds(start, T)],
                                  ssrc_v.at[slot], sems.at[slot]),
            pltpu.make_async_copy(dl_ref.at[pl.ds(start, T)],
                                  dl_v.at[slot], semd.at[slot]),
        )

    def out_copy(k, slot):
        return pltpu.make_async_copy(
            znew_v.at[slot], znew_ref.at[pl.ds(rs0 + k * T, T)],
            semo.at[slot])

    @pl.when(jnp.logical_and(b == 0, nch > 0))
    def _():
        for c in in_copies(0, 0):
            c.start()

    def chunk(k, carry):
        slot = lax.rem(k, 2)
        nslot = 1 - slot
        @pl.when(k + 1 < nch)
        def _():
            for c in in_copies(k + 1, nslot):
                c.start()
        for c in in_copies(k, slot):
            c.wait()

        zc = z_v[slot]
        ss = ssrc_v[slot]
        dl = dl_v[slot]
        eidx = lax.broadcasted_iota(jnp.int32, (T, 1), 0) + k * T
        valid = eidx < n_e
        hit = jnp.logical_and(
            dl == lax.broadcasted_iota(jnp.int32, (T, NB), 1), valid)
        Pt = hit.astype(jnp.bfloat16)

        sd = _bmm(Pt, s_blk16)

        ss_out = _bmm(ss, ws_pack[...])
        sd_out = _bmm(sd, wd_pack[...])
        zc_out = _bmm(zc, wzc[...])

        e1 = _gelu(ss_out[:, 0:D] + sd_out[:, 0:D] + zc_out + be1[...])
        zn = zc + _bmm(e1, we2[...]) + be2[...]
        @pl.when(k >= 2)
        def _():
            out_copy(k - 2, slot).wait()
        znew_v[slot] = zn
        out_copy(k, slot).start()

        zn_out = _bmm(zn, wzn_pack[...])

        a1 = _gelu(sd_out[:, D:2 * D] + ss_out[:, D:2 * D]
                   + zn_out[:, 0:D] + ba1[...])
        a2 = _gelu(_bmm(a1, wa2[...]) + ba2[...])
        aw = _bmm(a2, wa3[...]) + ba3[...]
        v1 = _gelu(ss_out[:, 2 * D:3 * D] + zn_out[:, D:2 * D] + bv1[...])
        v2 = _gelu(_bmm(v1, wv2[...]) + bv2[...])
        av = _bmm(v2, wv3[...]) + bv3[...]

        awm = jnp.where(valid, aw, -1e30)
        cmax = jnp.max(awm, axis=0, keepdims=True)
        mold = mx[...]
        mnew = jnp.maximum(mold, cmax)
        scale = jnp.exp(mold - mnew)
        mx[...] = mnew
        ew = jnp.exp(awm - mnew)
        ew16 = ew.astype(jnp.bfloat16)
        den[...] = den[...] * scale + _dgt0(Pt, ew16)
        ptw = jnp.concatenate(
            [Pt * ew16[:, h:h + 1] for h in range(NUM_HEADS)], axis=1)
        srow = jnp.concatenate(
            [jnp.broadcast_to(scale[0:1, h:h + 1], (NB, 1))
             for h in range(NUM_HEADS)], axis=0)
        U[...] = U[...] * srow + _dgt0(ptw, av.astype(jnp.bfloat16))
        return carry

    lax.fori_loop(0, nch, chunk, 0)

    @pl.when(b + 1 < NBLOCK)
    def _():
        rs0n = rs_ref[b + 1]

        @pl.when(rs_ref[b + 2] > rs0n)
        def _():
            start = rs0n
            pltpu.make_async_copy(z_ref.at[pl.ds(start, T)],
                                  z_v.at[0], semz.at[0]).start()
            pltpu.make_async_copy(ssrc_ref.at[pl.ds(start, T)],
                                  ssrc_v.at[0], sems.at[0]).start()
            pltpu.make_async_copy(dl_ref.at[pl.ds(start, T)],
                                  dl_v.at[0], semd.at[0]).start()

    @pl.when(nch >= 2)
    def _():
        out_copy(nch - 2, lax.rem(nch, 2)).wait()

    @pl.when(nch >= 1)
    def _():
        out_copy(nch - 1, lax.rem(nch - 1, 2)).wait()

    dd = den[...]
    attn = [U[h * NB:(h + 1) * NB, :] / (dd[:, h:h + 1] + 1e-16)
            for h in range(NUM_HEADS)]
    attn_cat = jnp.concatenate(attn, axis=1)
    so = s_blk + _bmm(attn_cat, wao[...]) + bao[...]
    f1 = _gelu(_bmm(so, wf1[...]) + bf1[...])
    snew_ref[...] = so + _bmm(f1, wf2[...]) + bf2[...]


_N_W = 22


def _layer_specs():
    in_specs = [
        pl.BlockSpec(memory_space=pltpu.SMEM),
        pl.BlockSpec((NB, D), lambda b: (b, 0)),
        pl.BlockSpec(memory_space=pl.ANY),
        pl.BlockSpec(memory_space=pl.ANY),
        pl.BlockSpec(memory_space=pl.ANY),
    ]
    out_specs = [
        pl.BlockSpec((NB, D), lambda b: (b, 0)),
        pl.BlockSpec(memory_space=pl.ANY),
    ]
    out_shapes = [
        jax.ShapeDtypeStruct((N_PAD, D), jnp.float32),
        jax.ShapeDtypeStruct((E_PAD, D), jnp.float32),
    ]
    scratch_shapes = [
        pltpu.VMEM((2, T, D), jnp.float32),
        pltpu.VMEM((2, T, D), jnp.float32),
        pltpu.VMEM((2, T, 1), jnp.int32),
        pltpu.VMEM((2, T, D), jnp.float32),
        pltpu.VMEM((NUM_HEADS * NB, D), jnp.float32),
        pltpu.VMEM((NB, HP), jnp.float32),
        pltpu.VMEM((1, HP), jnp.float32),
        pltpu.SemaphoreType.DMA((2,)),
        pltpu.SemaphoreType.DMA((2,)),
        pltpu.SemaphoreType.DMA((2,)),
        pltpu.SemaphoreType.DMA((2,)),
    ]
    return in_specs, out_specs, out_shapes, scratch_shapes


def _layer_call(row_start, s_pad, z_s, ssrc, dstloc, weights, interpret=False):
    in_specs, out_specs, out_shapes, scratch_shapes = _layer_specs()
    full_vmem = [pl.BlockSpec(w.shape, lambda b, _n=w.ndim: (0,) * _n)
                 for w in weights]
    fn = pl.pallas_call(
        _layer_body,
        grid=(NBLOCK,),
        in_specs=in_specs + full_vmem,
        out_specs=out_specs,
        out_shape=out_shapes,
        scratch_shapes=scratch_shapes,
        compiler_params=pltpu.CompilerParams(
            dimension_semantics=("arbitrary",)),
        interpret=interpret,
    )
    return fn(row_start, s_pad, z_s, ssrc, dstloc, *weights)


def _fold(p):
    g = 1.0 / np.sqrt(1.0 + 1e-5)

    def bnfold(w, bvec, gamma, beta):
        sc = gamma * g
        return w * sc[None, :], (bvec * sc + beta)[None, :]

    (ew1, eb1), (ew2, eb2) = p['eff']
    ew2f, eb2f = bnfold(ew2, eb2, p['eff_bn'][0], p['eff_bn'][1])
    (aw1, ab1), (aw2, ab2), (aw3, ab3) = p['aw']
    aw3p = jnp.zeros((D, HP), jnp.float32).at[:, :NUM_HEADS].set(aw3)
    ab3p = jnp.zeros((HP,), jnp.float32).at[:NUM_HEADS].set(ab3)
    (vw1, vb1), (vw2, vb2), (vw3, vb3) = p['av']
    ow, ob = p['ao']
    owf, obf = bnfold(ow, ob, p['ao_bn'][0], p['ao_bn'][1])
    (fw1, fb1), (fw2, fb2) = p['ffn']
    fw2f, fb2f = bnfold(fw2, fb2, p['ffn_bn'][0], p['ffn_bn'][1])
    ws_pack = jnp.concatenate([ew1[0:D], aw1[D:2 * D], vw1[0:D]], axis=1)
    wd_pack = jnp.concatenate([ew1[D:2 * D], aw1[0:D]], axis=1)
    wzc = ew1[2 * D:3 * D]
    wzn_pack = jnp.concatenate([aw1[2 * D:3 * D], vw1[D:2 * D]], axis=1)
    b16 = lambda w: w.astype(jnp.bfloat16)
    return [
        b16(ws_pack), b16(wd_pack), b16(wzc), eb1[None, :],
        b16(ew2f), eb2f,
        b16(wzn_pack), ab1[None, :], b16(aw2), ab2[None, :],
        b16(aw3p), ab3p[None, :],
        vb1[None, :], b16(vw2), vb2[None, :], b16(vw3), vb3[None, :],
        b16(owf), obf, b16(fw1), fb1[None, :], b16(fw2f), fb2f,
    ]


def kernel(s, z, edge_idx, params):
    src = edge_idx[0]
    dst = edge_idx[1]

    eiota = jnp.arange(N_EDGES, dtype=jnp.int32)
    dst_s, src_s, perm = lax.sort((dst, src, eiota), num_keys=1)
    pad_i = jnp.zeros((E_PAD - N_EDGES,), jnp.int32)
    src_p = jnp.concatenate([src_s, pad_i])
    perm_p = jnp.concatenate([perm, pad_i])
    perm_scat = jnp.concatenate(
        [perm, jnp.arange(N_EDGES, E_PAD, dtype=jnp.int32)])
    dstloc = jnp.concatenate([(dst_s % NB).astype(jnp.int32), pad_i])
    dstloc = dstloc.reshape(E_PAD, 1)
    bounds = jnp.arange(0, N_PAD + NB, NB, dtype=jnp.int32)
    row_start = jnp.searchsorted(dst_s, bounds).astype(jnp.int32)

    s_pad = jnp.zeros((N_PAD, D), jnp.float32).at[:N_NODES, :].set(s)
    z_s = _sc_gather(z, perm_p)

    for p in params:
        weights = _fold(p)
        ssrc = _sc_gather(s_pad, src_p)
        s_pad, z_s = _layer_call(row_start, s_pad, z_s, ssrc, dstloc, weights)

    z_out = _sc_scatter(z_s, perm_scat)[:N_EDGES]
    return (s_pad[:N_NODES], z_out)
